# baseline (speedup 1.0000x reference)
import functools

import jax
import jax.numpy as jnp
from jax import lax
from jax.experimental import pallas as pl
from jax.experimental.pallas import tpu as pltpu
from jax.experimental.pallas import tpu_sc as plsc

NC = 2
NS = 16
NW = NC * NS
K = 128
BLK = 16


def _sc_aggregate(h, src3, dst3, zeros, n_pad, cpt):
  D = h.shape[1]
  rpt = n_pad // NS

  mesh = plsc.VectorSubcoreMesh(core_axis_name="c", subcore_axis_name="s",
                                num_cores=NC, num_subcores=NS)

  @functools.partial(
      pl.kernel,
      out_type=[jax.ShapeDtypeStruct((n_pad, D), jnp.float32),
                jax.ShapeDtypeStruct((n_pad, D), jnp.float32)],
      mesh=mesh,
      scratch_types=[
          pltpu.VMEM((2, BLK, K), jnp.int32),
          pltpu.VMEM((2, BLK, K), jnp.int32),
          pltpu.VMEM((K, D), jnp.float32),
          pltpu.VMEM((K, D), jnp.float32),
          pltpu.VMEM_SHARED((n_pad, D), jnp.float32),
          pltpu.SemaphoreType.DMA,
          pltpu.SemaphoreType.DMA,
          pltpu.SemaphoreType.DMA,
          pltpu.SemaphoreType.DMA,
      ],
  )
  def agg_kernel(h_hbm, src_hbm, dst_hbm, zeros_hbm, out0_hbm, out1_hbm,
                 sidx, didx, rows_a, rows_b, acc_sh,
                 sem_a, sem_b, isem_s, isem_d):
    cid = lax.axis_index("c")
    sid = lax.axis_index("s")
    wid = cid * NS + sid
    nblk = cpt // BLK

    pltpu.sync_copy(zeros_hbm.at[pl.ds(sid * rpt, rpt)],
                    acc_sh.at[pl.ds(sid * rpt, rpt)])
    pltpu.sync_copy(src_hbm.at[wid, pl.ds(0, BLK)], sidx.at[0])
    pltpu.sync_copy(dst_hbm.at[wid, pl.ds(0, BLK)], didx.at[0])
    plsc.subcore_barrier()
    pltpu.async_copy(h_hbm.at[sidx.at[0, 0]], rows_a, sem_a)

    def blk_body(b, carry):
      p = b % 2
      pn = 1 - p

      @pl.when(b < nblk - 1)
      def _():
        pltpu.async_copy(src_hbm.at[wid, pl.ds((b + 1) * BLK, BLK)],
                         sidx.at[pn], isem_s)
        pltpu.async_copy(dst_hbm.at[wid, pl.ds((b + 1) * BLK, BLK)],
                         didx.at[pn], isem_d)

      for jj in range(BLK):
        rcur, scur = (rows_a, sem_a) if jj % 2 == 0 else (rows_b, sem_b)
        rnxt, snxt = (rows_b, sem_b) if jj % 2 == 0 else (rows_a, sem_a)
        pltpu.make_async_copy(h_hbm.at[sidx.at[p, jj]], rcur, scur).wait()
        if jj < BLK - 1:
          pltpu.async_copy(h_hbm.at[sidx.at[p, jj + 1]], rnxt, snxt)
        else:
          @pl.when(b < nblk - 1)
          def _():
            pltpu.make_async_copy(src_hbm.at[wid, pl.ds((b + 1) * BLK, BLK)],
                                  sidx.at[pn], isem_s).wait()
            pltpu.make_async_copy(dst_hbm.at[wid, pl.ds((b + 1) * BLK, BLK)],
                                  didx.at[pn], isem_d).wait()
            pltpu.async_copy(h_hbm.at[sidx.at[pn, 0]], rnxt, snxt)
        pltpu.sync_copy(rcur, acc_sh.at[didx.at[p, jj]], add=True)
      return carry

    lax.fori_loop(0, nblk, blk_body, 0, unroll=False)
    plsc.subcore_barrier()

    @pl.when(cid == 0)
    def _():
      pltpu.sync_copy(acc_sh.at[pl.ds(sid * rpt, rpt)],
                      out0_hbm.at[pl.ds(sid * rpt, rpt)])

    @pl.when(cid == 1)
    def _():
      pltpu.sync_copy(acc_sh.at[pl.ds(sid * rpt, rpt)],
                      out1_hbm.at[pl.ds(sid * rpt, rpt)])

  return agg_kernel(h, src3, dst3, zeros)


def _mlp_kernel(x_ref, a0_ref, a1_ref, wa_ref, ba_ref, wb_ref, bb_ref,
                o_ref, *, relu_out):
  z = x_ref[...] + a0_ref[...] + a1_ref[...]
  t = jnp.maximum(
      jnp.dot(z, wa_ref[...], preferred_element_type=jnp.float32)
      + ba_ref[...], 0.0)
  o = jnp.dot(t, wb_ref[...], preferred_element_type=jnp.float32) + bb_ref[...]
  if relu_out:
    o = jnp.maximum(o, 0.0)
  o_ref[...] = o


def _mlp(h, a0, a1, wa, ba, wb, bb, relu_out, rows):
  N, D = h.shape
  nblk = N // rows
  row_spec = pl.BlockSpec((rows, D), lambda i: (i, 0))
  full = pl.BlockSpec((D, D), lambda i: (0, 0))
  vec = pl.BlockSpec((1, D), lambda i: (0, 0))
  return pl.pallas_call(
      functools.partial(_mlp_kernel, relu_out=relu_out),
      grid=(nblk,),
      in_specs=[row_spec, row_spec, row_spec, full, vec, full, vec],
      out_specs=row_spec,
      out_shape=jax.ShapeDtypeStruct((N, D), jnp.float32),
  )(h, a0, a1, wa, ba.reshape(1, D), wb, bb.reshape(1, D))


def _mlp_pool_kernel(x_ref, a0_ref, a1_ref, wa_ref, ba_ref, wb_ref, bb_ref,
                     ids_ref, o_ref, acc_ref, cnt_ref, *, n_graphs, rows,
                     nblk):
  i = pl.program_id(0)

  @pl.when(i == 0)
  def _():
    acc_ref[...] = jnp.zeros_like(acc_ref)
    cnt_ref[...] = jnp.zeros_like(cnt_ref)

  z = x_ref[...] + a0_ref[...] + a1_ref[...]
  t = jnp.maximum(
      jnp.dot(z, wa_ref[...], preferred_element_type=jnp.float32)
      + ba_ref[...], 0.0)
  h2 = jnp.dot(t, wb_ref[...], preferred_element_type=jnp.float32) + bb_ref[...]

  ids = ids_ref[0]
  gid = lax.broadcasted_iota(jnp.int32, (n_graphs, rows), 0)
  onehot = (gid == ids).astype(jnp.float32)
  acc_ref[...] += jnp.dot(onehot, h2, preferred_element_type=jnp.float32)
  cnt_ref[...] += jnp.broadcast_to(
      jnp.sum(onehot, axis=1, keepdims=True), cnt_ref.shape)

  @pl.when(i == nblk - 1)
  def _():
    o_ref[...] = acc_ref[...] / jnp.maximum(cnt_ref[...], 1.0)


def _mlp_pool(h, a0, a1, wa, ba, wb, bb, ids3, n_graphs, rows):
  N, D = h.shape
  nblk = N // rows
  row_spec = pl.BlockSpec((rows, D), lambda i: (i, 0))
  full = pl.BlockSpec((D, D), lambda i: (0, 0))
  vec = pl.BlockSpec((1, D), lambda i: (0, 0))
  ids_spec = pl.BlockSpec((1, 1, rows), lambda i: (i, 0, 0))
  out_spec = pl.BlockSpec((n_graphs, D), lambda i: (0, 0))
  return pl.pallas_call(
      functools.partial(_mlp_pool_kernel, n_graphs=n_graphs, rows=rows,
                        nblk=nblk),
      grid=(nblk,),
      in_specs=[row_spec, row_spec, row_spec, full, vec, full, vec, ids_spec],
      out_specs=out_spec,
      out_shape=jax.ShapeDtypeStruct((n_graphs, D), jnp.float32),
      scratch_shapes=[
          pltpu.VMEM((n_graphs, D), jnp.float32),
          pltpu.VMEM((n_graphs, D), jnp.float32),
      ],
  )(h, a0, a1, wa, ba.reshape(1, D), wb, bb.reshape(1, D), ids3)


def kernel(x, edge_index, batch, W1a, b1a, W1b, b1b, W2a, b2a, W2b, b2b):
  N, D = x.shape
  E = edge_index.shape[1]
  n_graphs = 64
  rows = 1000

  n_pad = ((N + NW * 8 - 1) // (NW * 8)) * (NW * 8)
  if n_pad == N:
    n_pad = N + NW * 8
  ept = -(-E // NW)
  cpt = -(-ept // (K * BLK)) * BLK
  e_pad = NW * cpt * K

  src = edge_index[0].astype(jnp.int32)
  dst = edge_index[1].astype(jnp.int32)
  pad_src = jnp.arange(e_pad - E, dtype=jnp.int32) % N
  src3 = jnp.concatenate([src, pad_src]).reshape(NW, cpt, K)
  pad_dst = N + jnp.arange(e_pad - E, dtype=jnp.int32) % (n_pad - N)
  dst3 = jnp.concatenate([dst, pad_dst]).reshape(NW, cpt, K)
  zeros = jnp.zeros((n_pad, D), jnp.float32)
  ids3 = batch.astype(jnp.int32).reshape(N // rows, 1, rows)

  a1_0, a1_1 = _sc_aggregate(x, src3, dst3, zeros, n_pad, cpt)
  h1 = _mlp(x, a1_0, a1_1, W1a, b1a, W1b, b1b, relu_out=True, rows=rows)
  a2_0, a2_1 = _sc_aggregate(h1, src3, dst3, zeros, n_pad, cpt)
  return _mlp_pool(h1, a2_0, a2_1, W2a, b2a, W2b, b2b, ids3, n_graphs, rows)


# speedup vs baseline: 1.0090x; 1.0090x over previous
import functools

import jax
import jax.numpy as jnp
from jax import lax
from jax.experimental import pallas as pl
from jax.experimental.pallas import tpu as pltpu
from jax.experimental.pallas import tpu_sc as plsc

NC = 2
NS = 16
NW = NC * NS
K = 128
BLK = 8


def _sc_aggregate(h, src3, dst3, zeros, n_pad, cpt):
  D = h.shape[1]
  rpt = n_pad // NS

  mesh = plsc.VectorSubcoreMesh(core_axis_name="c", subcore_axis_name="s",
                                num_cores=NC, num_subcores=NS)

  @functools.partial(
      pl.kernel,
      out_type=[jax.ShapeDtypeStruct((n_pad, D), jnp.float32),
                jax.ShapeDtypeStruct((n_pad, D), jnp.float32)],
      mesh=mesh,
      scratch_types=[
          pltpu.VMEM((2, BLK, K), jnp.int32),
          pltpu.VMEM((2, BLK, K), jnp.int32),
          pltpu.VMEM((K, D), jnp.float32),
          pltpu.VMEM((K, D), jnp.float32),
          pltpu.VMEM_SHARED((n_pad, D), jnp.float32),
          pltpu.SemaphoreType.DMA,
          pltpu.SemaphoreType.DMA,
          pltpu.SemaphoreType.DMA,
          pltpu.SemaphoreType.DMA,
      ],
  )
  def agg_kernel(h_hbm, src_hbm, dst_hbm, zeros_hbm, out0_hbm, out1_hbm,
                 sidx, didx, rows_a, rows_b, acc_sh,
                 sem_a, sem_b, isem_s, isem_d):
    cid = lax.axis_index("c")
    sid = lax.axis_index("s")
    wid = cid * NS + sid
    nblk = cpt // BLK

    pltpu.sync_copy(zeros_hbm.at[pl.ds(sid * rpt, rpt)],
                    acc_sh.at[pl.ds(sid * rpt, rpt)])
    pltpu.sync_copy(src_hbm.at[wid, pl.ds(0, BLK)], sidx.at[0])
    pltpu.sync_copy(dst_hbm.at[wid, pl.ds(0, BLK)], didx.at[0])
    plsc.subcore_barrier()
    pltpu.async_copy(h_hbm.at[sidx.at[0, 0]], rows_a, sem_a)

    def blk_body(b, carry):
      p = b % 2
      pn = 1 - p

      @pl.when(b < nblk - 1)
      def _():
        pltpu.async_copy(src_hbm.at[wid, pl.ds((b + 1) * BLK, BLK)],
                         sidx.at[pn], isem_s)
        pltpu.async_copy(dst_hbm.at[wid, pl.ds((b + 1) * BLK, BLK)],
                         didx.at[pn], isem_d)

      for jj in range(BLK):
        rcur, scur = (rows_a, sem_a) if jj % 2 == 0 else (rows_b, sem_b)
        rnxt, snxt = (rows_b, sem_b) if jj % 2 == 0 else (rows_a, sem_a)
        pltpu.make_async_copy(h_hbm.at[sidx.at[p, jj]], rcur, scur).wait()
        if jj < BLK - 1:
          pltpu.async_copy(h_hbm.at[sidx.at[p, jj + 1]], rnxt, snxt)
        else:
          @pl.when(b < nblk - 1)
          def _():
            pltpu.make_async_copy(src_hbm.at[wid, pl.ds((b + 1) * BLK, BLK)],
                                  sidx.at[pn], isem_s).wait()
            pltpu.make_async_copy(dst_hbm.at[wid, pl.ds((b + 1) * BLK, BLK)],
                                  didx.at[pn], isem_d).wait()
            pltpu.async_copy(h_hbm.at[sidx.at[pn, 0]], rnxt, snxt)
        pltpu.sync_copy(rcur, acc_sh.at[didx.at[p, jj]], add=True)
      return carry

    lax.fori_loop(0, nblk, blk_body, 0, unroll=False)
    plsc.subcore_barrier()

    @pl.when(cid == 0)
    def _():
      pltpu.sync_copy(acc_sh.at[pl.ds(sid * rpt, rpt)],
                      out0_hbm.at[pl.ds(sid * rpt, rpt)])

    @pl.when(cid == 1)
    def _():
      pltpu.sync_copy(acc_sh.at[pl.ds(sid * rpt, rpt)],
                      out1_hbm.at[pl.ds(sid * rpt, rpt)])

  return agg_kernel(h, src3, dst3, zeros)


def _mlp_kernel(x_ref, a0_ref, a1_ref, wa_ref, ba_ref, wb_ref, bb_ref,
                o_ref, *, relu_out):
  z = x_ref[...] + a0_ref[...] + a1_ref[...]
  t = jnp.maximum(
      jnp.dot(z, wa_ref[...], preferred_element_type=jnp.float32)
      + ba_ref[...], 0.0)
  o = jnp.dot(t, wb_ref[...], preferred_element_type=jnp.float32) + bb_ref[...]
  if relu_out:
    o = jnp.maximum(o, 0.0)
  o_ref[...] = o


def _mlp(h, a0, a1, wa, ba, wb, bb, relu_out, rows):
  N, D = h.shape
  nblk = N // rows
  row_spec = pl.BlockSpec((rows, D), lambda i: (i, 0))
  full = pl.BlockSpec((D, D), lambda i: (0, 0))
  vec = pl.BlockSpec((1, D), lambda i: (0, 0))
  return pl.pallas_call(
      functools.partial(_mlp_kernel, relu_out=relu_out),
      grid=(nblk,),
      in_specs=[row_spec, row_spec, row_spec, full, vec, full, vec],
      out_specs=row_spec,
      out_shape=jax.ShapeDtypeStruct((N, D), jnp.float32),
  )(h, a0, a1, wa, ba.reshape(1, D), wb, bb.reshape(1, D))


def _mlp_pool_kernel(x_ref, a0_ref, a1_ref, wa_ref, ba_ref, wb_ref, bb_ref,
                     ids_ref, o_ref, acc_ref, cnt_ref, *, n_graphs, rows,
                     nblk):
  i = pl.program_id(0)

  @pl.when(i == 0)
  def _():
    acc_ref[...] = jnp.zeros_like(acc_ref)
    cnt_ref[...] = jnp.zeros_like(cnt_ref)

  z = x_ref[...] + a0_ref[...] + a1_ref[...]
  t = jnp.maximum(
      jnp.dot(z, wa_ref[...], preferred_element_type=jnp.float32)
      + ba_ref[...], 0.0)
  h2 = jnp.dot(t, wb_ref[...], preferred_element_type=jnp.float32) + bb_ref[...]

  ids = ids_ref[0]
  gid = lax.broadcasted_iota(jnp.int32, (n_graphs, rows), 0)
  onehot = (gid == ids).astype(jnp.float32)
  acc_ref[...] += jnp.dot(onehot, h2, preferred_element_type=jnp.float32)
  cnt_ref[...] += jnp.broadcast_to(
      jnp.sum(onehot, axis=1, keepdims=True), cnt_ref.shape)

  @pl.when(i == nblk - 1)
  def _():
    o_ref[...] = acc_ref[...] / jnp.maximum(cnt_ref[...], 1.0)


def _mlp_pool(h, a0, a1, wa, ba, wb, bb, ids3, n_graphs, rows):
  N, D = h.shape
  nblk = N // rows
  row_spec = pl.BlockSpec((rows, D), lambda i: (i, 0))
  full = pl.BlockSpec((D, D), lambda i: (0, 0))
  vec = pl.BlockSpec((1, D), lambda i: (0, 0))
  ids_spec = pl.BlockSpec((1, 1, rows), lambda i: (i, 0, 0))
  out_spec = pl.BlockSpec((n_graphs, D), lambda i: (0, 0))
  return pl.pallas_call(
      functools.partial(_mlp_pool_kernel, n_graphs=n_graphs, rows=rows,
                        nblk=nblk),
      grid=(nblk,),
      in_specs=[row_spec, row_spec, row_spec, full, vec, full, vec, ids_spec],
      out_specs=out_spec,
      out_shape=jax.ShapeDtypeStruct((n_graphs, D), jnp.float32),
      scratch_shapes=[
          pltpu.VMEM((n_graphs, D), jnp.float32),
          pltpu.VMEM((n_graphs, D), jnp.float32),
      ],
  )(h, a0, a1, wa, ba.reshape(1, D), wb, bb.reshape(1, D), ids3)


def kernel(x, edge_index, batch, W1a, b1a, W1b, b1b, W2a, b2a, W2b, b2b):
  N, D = x.shape
  E = edge_index.shape[1]
  n_graphs = 64
  rows = 1000

  n_pad = ((N + NW * 8 - 1) // (NW * 8)) * (NW * 8)
  if n_pad == N:
    n_pad = N + NW * 8
  ept = -(-E // NW)
  cpt = -(-ept // (K * BLK)) * BLK
  e_pad = NW * cpt * K

  src = edge_index[0].astype(jnp.int32)
  dst = edge_index[1].astype(jnp.int32)
  pad_src = jnp.arange(e_pad - E, dtype=jnp.int32) % N
  src3 = jnp.concatenate([src, pad_src]).reshape(NW, cpt, K)
  pad_dst = N + jnp.arange(e_pad - E, dtype=jnp.int32) % (n_pad - N)
  dst3 = jnp.concatenate([dst, pad_dst]).reshape(NW, cpt, K)
  zeros = jnp.zeros((n_pad, D), jnp.float32)
  ids3 = batch.astype(jnp.int32).reshape(N // rows, 1, rows)

  a1_0, a1_1 = _sc_aggregate(x, src3, dst3, zeros, n_pad, cpt)
  h1 = _mlp(x, a1_0, a1_1, W1a, b1a, W1b, b1b, relu_out=True, rows=rows)
  a2_0, a2_1 = _sc_aggregate(h1, src3, dst3, zeros, n_pad, cpt)
  return _mlp_pool(h1, a2_0, a2_1, W2a, b2a, W2b, b2b, ids3, n_graphs, rows)
